# baseline (speedup 1.0000x reference)
import functools

import jax
import jax.numpy as jnp
from jax import lax
from jax.experimental import pallas as pl
from jax.experimental.pallas import tpu as pltpu
from jax.experimental.pallas import tpu_sc as plsc

N = 10000
D = 256
DH = 128
E = 160000
NC = 2
NS = 16
NW = NC * NS
ER = E // 128
CHR = 32
NCHK = 3
UPC = 2 * CHR
ERP = ER + 30
OWN = 640
ZR = 40
RB = 1000
F32 = jnp.float32

_sc_params = pltpu.CompilerParams(use_tc_tiling_on_sc=False,
                                  needs_layout_passes=False)
_mesh = functools.partial(
    plsc.VectorSubcoreMesh, core_axis_name="c", subcore_axis_name="s")


def _zero_vmem_2d(ref, nrows, ncols):
    z = jnp.zeros((16,), F32)

    def body(i, _):
        for j in range(ncols // 16):
            ref[i, pl.ds(j * 16, 16)] = z
        return 0

    lax.fori_loop(0, nrows, body, 0)


def _deg_body(col2d, ew2d, d0, d1, cbig, ebig, bufA, bufB, zbuf, acc, ssem):
    c = lax.axis_index("c")
    s = lax.axis_index("s")
    w = s * NC + c

    _zero_vmem_2d(zbuf, ZR, 16)
    _zero_vmem_2d(bufA, 128, 16)
    _zero_vmem_2d(bufB, 128, 16)
    lo_own = s * OWN
    ncop = jnp.where(s == NS - 1, (N - (NS - 1) * OWN) // ZR, OWN // ZR)

    def zcp(t, _):
        pltpu.sync_copy(zbuf, acc.at[pl.ds(lo_own + t * ZR, ZR)])
        return 0

    lax.fori_loop(0, ncop, zcp, 0)
    plsc.subcore_barrier()

    lo = (w * ER) // NW
    nloc = ((w + 1) * ER) // NW - lo
    pltpu.sync_copy(col2d.at[pl.ds(lo, 40)], cbig)
    pltpu.sync_copy(ew2d.at[pl.ds(lo, 40)], ebig)
    lane = lax.iota(jnp.int32, 16)
    zlane = jnp.zeros((16,), jnp.int32)

    def drain():
        pltpu.make_async_copy(bufA, acc.at[cbig.at[0]], ssem).wait()

    def pair_body(g, _):
        for b in range(2):
            buf = bufA if b == 0 else bufB
            rl = 2 * g + b

            @pl.when(rl < nloc)
            def _():
                @pl.when(rl >= 2)
                def _():
                    drain()

                for q in range(8):
                    ew16 = ebig[rl, pl.ds(q * 16, 16)]
                    plsc.store_scatter(buf, [lane + q * 16, zlane], ew16)
                pltpu.async_copy(buf, acc.at[cbig.at[rl]], ssem, add=True)

        return 0

    lax.fori_loop(0, 20, pair_body, 0)
    drain()
    drain()
    plsc.subcore_barrier()

    def wb(t, _):
        sl = pl.ds(lo_own + t * ZR, ZR)

        @pl.when(c == 0)
        def _():
            pltpu.sync_copy(acc.at[sl], d0.at[sl])

        @pl.when(c == 1)
        def _():
            pltpu.sync_copy(acc.at[sl], d1.at[sl])

        return 0

    lax.fori_loop(0, ncop, wb, 0)


_deg_call = pl.kernel(
    _deg_body,
    out_type=(jax.ShapeDtypeStruct((N, 16), F32),
              jax.ShapeDtypeStruct((N, 16), F32)),
    mesh=_mesh(),
    scratch_types=[
        pltpu.VMEM((40, 128), jnp.int32),
        pltpu.VMEM((40, 128), F32),
        pltpu.VMEM((128, 16), F32),
        pltpu.VMEM((128, 16), F32),
        pltpu.VMEM((ZR, 16), F32),
        pltpu.VMEM_SHARED((N, 16), F32),
        pltpu.SemaphoreType.DMA,
    ],
    compiler_params=_sc_params,
)


def _spmm_body(row2d, col2d, ew2d, h0, h1, out0, out1,
               rbig, cbig, ebig, r0, r1, r2, r3, i0, i1, i2, i3,
               acc, gsem, ssem):
    c = lax.axis_index("c")
    s = lax.axis_index("s")
    R = (r0, r1, r2, r3)
    CIDX = (i0, i1, i2, i3)

    _zero_vmem_2d(r0, ZR, DH)
    lo_own = s * OWN
    ncop = jnp.where(s == NS - 1, (N - (NS - 1) * OWN) // ZR, OWN // ZR)

    def zcp(t, _):
        pltpu.sync_copy(r0.at[pl.ds(0, ZR)],
                        acc.at[pl.ds(lo_own + t * ZR, ZR)])
        return 0

    lax.fori_loop(0, ncop, zcp, 0)
    plsc.subcore_barrier()

    lo = (s * ER) // NS
    nrows = ((s + 1) * ER) // NS - lo
    nunits = 2 * nrows

    def issue_gather(idx, dst):
        @pl.when(c == 0)
        def _():
            pltpu.async_copy(h0.at[idx], dst, gsem)

        @pl.when(c == 1)
        def _():
            pltpu.async_copy(h1.at[idx], dst, gsem)

    def gidx(k, h):
        return rbig.at[k, pl.ds(64 * h, 64)]

    def wait_gather(dst):
        pltpu.make_async_copy(h0.at[gidx(0, 0)], dst, gsem).wait()

    def drain_scatter():
        pltpu.make_async_copy(r0, acc.at[i0], ssem).wait()

    def scale(k, h, dst):
        def scale_q(q, _):
            for k2 in range(32):
                kk = q * 32 + k2
                spl = plsc.load_gather(
                    ebig, [jnp.full((16,), k, jnp.int32),
                           jnp.full((16,), 64 * h + kk, jnp.int32)])
                for j in range(DH // 16):
                    sl = pl.ds(j * 16, 16)
                    dst[kk, sl] = dst[kk, sl] * spl
            return 0

        lax.fori_loop(0, 2, scale_q, 0)

    def chunk_body(ci, _):
        pltpu.sync_copy(row2d.at[pl.ds(lo + ci * CHR, CHR)], rbig)
        pltpu.sync_copy(col2d.at[pl.ds(lo + ci * CHR, CHR)], cbig)
        pltpu.sync_copy(ew2d.at[pl.ds(lo + ci * CHR, CHR)], ebig)
        issue_gather(gidx(0, 0), r0)
        issue_gather(gidx(0, 1), r1)

        def group_body(g, _):
            for uu in range(4):
                ul = 4 * g + uu
                u = ci * UPC + ul
                k = 2 * g + uu // 2
                h = uu % 2
                buf = R[uu]
                cix = CIDX[uu]

                @pl.when(u < nunits)
                def _():
                    @pl.when(u >= 2)
                    def _():
                        drain_scatter()

                    pre_ok = (u + 2 < nunits) if uu < 2 else (
                        (g < CHR // 2 - 1) & (u + 2 < nunits))

                    @pl.when(pre_ok)
                    def _():
                        issue_gather(gidx(2 * g + (uu + 2) // 2, h),
                                     R[(uu + 2) % 4])

                    wait_gather(buf)
                    scale(k, h, buf)
                    for q in range(4):
                        cix[pl.ds(q * 16, 16)] = (
                            cbig[k, pl.ds(64 * h + q * 16, 16)])
                    pltpu.async_copy(buf, acc.at[cix], ssem, add=True)

            return 0

        lax.fori_loop(0, CHR // 2, group_body, 0)
        return 0

    lax.fori_loop(0, NCHK, chunk_body, 0)
    drain_scatter()
    drain_scatter()
    plsc.subcore_barrier()

    def wb(t, _):
        sl = pl.ds(lo_own + t * ZR, ZR)

        @pl.when(c == 0)
        def _():
            pltpu.sync_copy(acc.at[sl], out0.at[sl])

        @pl.when(c == 1)
        def _():
            pltpu.sync_copy(acc.at[sl], out1.at[sl])

        return 0

    lax.fori_loop(0, ncop, wb, 0)


_spmm_call = pl.kernel(
    _spmm_body,
    out_type=(jax.ShapeDtypeStruct((N, DH), F32),
              jax.ShapeDtypeStruct((N, DH), F32)),
    mesh=_mesh(),
    scratch_types=[
        pltpu.VMEM((CHR, 128), jnp.int32),
        pltpu.VMEM((CHR, 128), jnp.int32),
        pltpu.VMEM((CHR, 128), F32),
        pltpu.VMEM((64, DH), F32),
        pltpu.VMEM((64, DH), F32),
        pltpu.VMEM((64, DH), F32),
        pltpu.VMEM((64, DH), F32),
        pltpu.VMEM((64,), jnp.int32),
        pltpu.VMEM((64,), jnp.int32),
        pltpu.VMEM((64,), jnp.int32),
        pltpu.VMEM((64,), jnp.int32),
        pltpu.VMEM_SHARED((N, DH), F32),
        pltpu.SemaphoreType.DMA,
        pltpu.SemaphoreType.DMA,
    ],
    compiler_params=_sc_params,
)


def _blk(shape, imap):
    return pl.BlockSpec(shape, imap)


_row_map = lambda i: (i, 0)
_fix_map = lambda i: (0, 0)


def _dis_body(d0_ref, d1_ref, dis_ref):
    deg = d0_ref[...][:, 0:1] + d1_ref[...][:, 0:1] + 1.0
    dis = jnp.where(deg > 0, lax.rsqrt(jnp.where(deg > 0, deg, 1.0)), 0.0)
    dis_ref[...] = jnp.broadcast_to(dis, (RB, DH))


_dis_call = pl.pallas_call(
    _dis_body,
    grid=(N // RB,),
    in_specs=[_blk((RB, 16), _row_map), _blk((RB, 16), _row_map)],
    out_specs=_blk((RB, DH), _row_map),
    out_shape=jax.ShapeDtypeStruct((N, DH), F32),
)


def _m0_body(x_ref, w_ref, dis_ref, h0_ref, h1_ref):
    dis = dis_ref[...][:, 0:1]
    h = jnp.dot(x_ref[...], w_ref[...], preferred_element_type=F32,
                precision=lax.Precision.HIGHEST) * dis
    h0_ref[...] = h[:, :DH]
    h1_ref[...] = h[:, DH:]


_m0_call = pl.pallas_call(
    _m0_body,
    grid=(N // RB,),
    in_specs=[
        _blk((RB, D), _row_map),
        _blk((D, D), _fix_map),
        _blk((RB, DH), _row_map),
    ],
    out_specs=[_blk((RB, DH), _row_map), _blk((RB, DH), _row_map)],
    out_shape=(jax.ShapeDtypeStruct((N, DH), F32),
               jax.ShapeDtypeStruct((N, DH), F32)),
)


def _asm_body(has_res, *refs):
    if has_res:
        (s0, s1, h0, h1, dis_ref, b, res, t_ref, sum_ref, sq_ref) = refs
    else:
        (s0, s1, h0, h1, dis_ref, b, t_ref, sum_ref, sq_ref) = refs
        res = None
    dis = dis_ref[...][:, 0:1]
    t = jnp.concatenate([s0[...] + h0[...], s1[...] + h1[...]], axis=1)
    t = t * dis + b[...]
    if res is not None:
        t = t + res[...]
    t_ref[...] = t

    @pl.when(pl.program_id(0) == 0)
    def _():
        sum_ref[...] = jnp.zeros_like(sum_ref)
        sq_ref[...] = jnp.zeros_like(sq_ref)

    sum_ref[...] += jnp.sum(t, axis=0, keepdims=True)
    sq_ref[...] += jnp.sum(t * t, axis=0, keepdims=True)


def _make_asm(has_res):
    in_specs = [
        _blk((RB, DH), _row_map),
        _blk((RB, DH), _row_map),
        _blk((RB, DH), _row_map),
        _blk((RB, DH), _row_map),
        _blk((RB, DH), _row_map),
        _blk((1, D), _fix_map),
    ]
    if has_res:
        in_specs.append(_blk((RB, D), _row_map))
    return pl.pallas_call(
        functools.partial(_asm_body, has_res),
        grid=(N // RB,),
        in_specs=in_specs,
        out_specs=[_blk((RB, D), _row_map), _blk((1, D), _fix_map),
                   _blk((1, D), _fix_map)],
        out_shape=(jax.ShapeDtypeStruct((N, D), F32),
                   jax.ShapeDtypeStruct((1, D), F32),
                   jax.ShapeDtypeStruct((1, D), F32)),
    )


_asm_call = _make_asm(False)
_asm_res_call = _make_asm(True)


def _bn_act(t_ref, sum_ref, sq_ref, g_ref, be_ref):
    mu = sum_ref[...] / N
    var = sq_ref[...] / N - mu * mu
    sc = lax.rsqrt(var + 1e-5) * g_ref[...]
    return jax.nn.relu((t_ref[...] - mu) * sc + be_ref[...])


def _bnmm_body(keep_act, *refs):
    if keep_act:
        (t_ref, sum_ref, sq_ref, g_ref, be_ref, w_ref, dis_ref,
         h0_ref, h1_ref, act_ref) = refs
    else:
        (t_ref, sum_ref, sq_ref, g_ref, be_ref, w_ref, dis_ref,
         h0_ref, h1_ref) = refs
        act_ref = None
    act = _bn_act(t_ref, sum_ref, sq_ref, g_ref, be_ref)
    dis = dis_ref[...][:, 0:1]
    h = jnp.dot(act, w_ref[...], preferred_element_type=F32,
                precision=lax.Precision.HIGHEST) * dis
    h0_ref[...] = h[:, :DH]
    h1_ref[...] = h[:, DH:]
    if act_ref is not None:
        act_ref[...] = act


def _make_bnmm(keep_act):
    out_specs = [_blk((RB, DH), _row_map), _blk((RB, DH), _row_map)]
    out_shape = [jax.ShapeDtypeStruct((N, DH), F32),
                 jax.ShapeDtypeStruct((N, DH), F32)]
    if keep_act:
        out_specs.append(_blk((RB, D), _row_map))
        out_shape.append(jax.ShapeDtypeStruct((N, D), F32))
    return pl.pallas_call(
        functools.partial(_bnmm_body, keep_act),
        grid=(N // RB,),
        in_specs=[
            _blk((RB, D), _row_map),
            _blk((1, D), _fix_map),
            _blk((1, D), _fix_map),
            _blk((1, D), _fix_map),
            _blk((1, D), _fix_map),
            _blk((D, D), _fix_map),
            _blk((RB, DH), _row_map),
        ],
        out_specs=out_specs,
        out_shape=tuple(out_shape),
    )


_bnmm_call = _make_bnmm(False)
_bnmm_act_call = _make_bnmm(True)


def _bnfinal_body(t_ref, sum_ref, sq_ref, g_ref, be_ref, o_ref):
    o_ref[...] = _bn_act(t_ref, sum_ref, sq_ref, g_ref, be_ref)


_bnfinal_call = pl.pallas_call(
    _bnfinal_body,
    grid=(N // RB,),
    in_specs=[
        _blk((RB, D), _row_map),
        _blk((1, D), _fix_map),
        _blk((1, D), _fix_map),
        _blk((1, D), _fix_map),
        _blk((1, D), _fix_map),
    ],
    out_specs=_blk((RB, D), _row_map),
    out_shape=jax.ShapeDtypeStruct((N, D), F32),
)


def kernel(x, edge_index, edge_attr, W0, b0, g0, be0, Wc0, bc0, gc0, bec0,
           Wc1, bc1, gc1, bec1, W1, b1, g1, be1):
    pad = ((0, ERP - ER), (0, 0))
    row2d = jnp.pad(edge_index[0].reshape(ER, 128), pad)
    col2d = jnp.pad(edge_index[1].reshape(ER, 128), pad)
    ew2d = jnp.pad(edge_attr.reshape(ER, 128), pad)
    r2 = lambda v: v.reshape(1, D)

    d0, d1 = _deg_call(col2d, ew2d)
    dis = _dis_call(d0, d1)
    h0, h1 = _m0_call(x, W0, dis)

    s0, s1 = _spmm_call(row2d, col2d, ew2d, h0, h1)
    t, sm, sq = _asm_call(s0, s1, h0, h1, dis, r2(b0))
    h0, h1, act0 = _bnmm_act_call(t, sm, sq, r2(g0), r2(be0), Wc0, dis)

    s0, s1 = _spmm_call(row2d, col2d, ew2d, h0, h1)
    t, sm, sq = _asm_res_call(s0, s1, h0, h1, dis, r2(bc0), act0)
    h0, h1 = _bnmm_call(t, sm, sq, r2(gc0), r2(bec0), Wc1, dis)

    s0, s1 = _spmm_call(row2d, col2d, ew2d, h0, h1)
    t, sm, sq = _asm_res_call(s0, s1, h0, h1, dis, r2(bc1), act0)
    h0, h1 = _bnmm_call(t, sm, sq, r2(gc1), r2(bec1), W1, dis)

    s0, s1 = _spmm_call(row2d, col2d, ew2d, h0, h1)
    t, sm, sq = _asm_call(s0, s1, h0, h1, dis, r2(b1))
    return _bnfinal_call(t, sm, sq, r2(g1), r2(be1))


# speedup vs baseline: 1.3493x; 1.3493x over previous
import functools

import jax
import jax.numpy as jnp
from jax import lax
from jax.experimental import pallas as pl
from jax.experimental.pallas import tpu as pltpu
from jax.experimental.pallas import tpu_sc as plsc

N = 10000
D = 256
DH = 128
E = 160000
NC = 2
NS = 16
NW = NC * NS
ER = E // 128
CHR = 32
NCHK = 3
UPC = 2 * CHR
ERP = ER + 30
OWN = 640
ZR = 40
RB = 1000
F32 = jnp.float32

_sc_params = pltpu.CompilerParams(use_tc_tiling_on_sc=False,
                                  needs_layout_passes=False)
_mesh = functools.partial(
    plsc.VectorSubcoreMesh, core_axis_name="c", subcore_axis_name="s")


def _zero_vmem_2d(ref, nrows, ncols):
    z = jnp.zeros((16,), F32)

    def body(i, _):
        for j in range(ncols // 16):
            ref[i, pl.ds(j * 16, 16)] = z
        return 0

    lax.fori_loop(0, nrows, body, 0)


def _deg_body(col2d, ew2d, d0, d1, cbig, ebig, bufA, bufB, zbuf, acc, ssem):
    c = lax.axis_index("c")
    s = lax.axis_index("s")
    w = s * NC + c

    _zero_vmem_2d(zbuf, ZR, 16)
    _zero_vmem_2d(bufA, 128, 16)
    _zero_vmem_2d(bufB, 128, 16)
    lo_own = s * OWN
    ncop = jnp.where(s == NS - 1, (N - (NS - 1) * OWN) // ZR, OWN // ZR)

    def zcp(t, _):
        pltpu.sync_copy(zbuf, acc.at[pl.ds(lo_own + t * ZR, ZR)])
        return 0

    lax.fori_loop(0, ncop, zcp, 0)
    plsc.subcore_barrier()

    lo = (w * ER) // NW
    nloc = ((w + 1) * ER) // NW - lo
    pltpu.sync_copy(col2d.at[pl.ds(lo, 40)], cbig)
    pltpu.sync_copy(ew2d.at[pl.ds(lo, 40)], ebig)
    lane = lax.iota(jnp.int32, 16)
    zlane = jnp.zeros((16,), jnp.int32)

    def drain():
        pltpu.make_async_copy(bufA, acc.at[cbig.at[0]], ssem).wait()

    def pair_body(g, _):
        for b in range(2):
            buf = bufA if b == 0 else bufB
            rl = 2 * g + b

            @pl.when(rl < nloc)
            def _():
                @pl.when(rl >= 2)
                def _():
                    drain()

                for q in range(8):
                    ew16 = ebig[rl, pl.ds(q * 16, 16)]
                    plsc.store_scatter(buf, [lane + q * 16, zlane], ew16)
                pltpu.async_copy(buf, acc.at[cbig.at[rl]], ssem, add=True)

        return 0

    lax.fori_loop(0, 20, pair_body, 0)
    drain()
    drain()
    plsc.subcore_barrier()

    def wb(t, _):
        sl = pl.ds(lo_own + t * ZR, ZR)

        @pl.when(c == 0)
        def _():
            pltpu.sync_copy(acc.at[sl], d0.at[sl])

        @pl.when(c == 1)
        def _():
            pltpu.sync_copy(acc.at[sl], d1.at[sl])

        return 0

    lax.fori_loop(0, ncop, wb, 0)


_deg_call = pl.kernel(
    _deg_body,
    out_type=(jax.ShapeDtypeStruct((N, 16), F32),
              jax.ShapeDtypeStruct((N, 16), F32)),
    mesh=_mesh(),
    scratch_types=[
        pltpu.VMEM((40, 128), jnp.int32),
        pltpu.VMEM((40, 128), F32),
        pltpu.VMEM((128, 16), F32),
        pltpu.VMEM((128, 16), F32),
        pltpu.VMEM((ZR, 16), F32),
        pltpu.VMEM_SHARED((N, 16), F32),
        pltpu.SemaphoreType.DMA,
    ],
    compiler_params=_sc_params,
)


def _spmm_body(row2d, col2d, ew2d, h0, h1, out0, out1,
               rbig, cbig, ebig, r0, r1, r2, r3, i0, i1, i2, i3,
               acc, gsem, ssem):
    c = lax.axis_index("c")
    s = lax.axis_index("s")
    R = (r0, r1, r2, r3)
    CIDX = (i0, i1, i2, i3)

    _zero_vmem_2d(r0, ZR, DH)
    lo_own = s * OWN
    ncop = jnp.where(s == NS - 1, (N - (NS - 1) * OWN) // ZR, OWN // ZR)

    def zcp(t, _):
        pltpu.sync_copy(r0.at[pl.ds(0, ZR)],
                        acc.at[pl.ds(lo_own + t * ZR, ZR)])
        return 0

    lax.fori_loop(0, ncop, zcp, 0)
    plsc.subcore_barrier()

    lo = (s * ER) // NS
    nrows = ((s + 1) * ER) // NS - lo
    nunits = 2 * nrows

    def issue_gather(idx, dst):
        @pl.when(c == 0)
        def _():
            pltpu.async_copy(h0.at[idx], dst, gsem)

        @pl.when(c == 1)
        def _():
            pltpu.async_copy(h1.at[idx], dst, gsem)

    def gidx(k, h):
        return rbig.at[k, pl.ds(64 * h, 64)]

    def wait_gather(dst):
        pltpu.make_async_copy(h0.at[gidx(0, 0)], dst, gsem).wait()

    def drain_scatter():
        pltpu.make_async_copy(r0, acc.at[i0], ssem).wait()

    def scale(k, h, dst):
        def scale_q(q, _):
            for k2 in range(16):
                kk = q * 16 + k2
                spl = plsc.load_gather(
                    ebig, [jnp.full((16,), k, jnp.int32),
                           jnp.full((16,), 64 * h + kk, jnp.int32)])
                for j in range(DH // 16):
                    sl = pl.ds(j * 16, 16)
                    dst[kk, sl] = dst[kk, sl] * spl
            return 0

        lax.fori_loop(0, 4, scale_q, 0)

    def chunk_body(ci, _):
        pltpu.sync_copy(row2d.at[pl.ds(lo + ci * CHR, CHR)], rbig)
        pltpu.sync_copy(col2d.at[pl.ds(lo + ci * CHR, CHR)], cbig)
        pltpu.sync_copy(ew2d.at[pl.ds(lo + ci * CHR, CHR)], ebig)
        issue_gather(gidx(0, 0), r0)
        issue_gather(gidx(0, 1), r1)

        def group_body(g, _):
            for uu in range(4):
                ul = 4 * g + uu
                u = ci * UPC + ul
                k = 2 * g + uu // 2
                h = uu % 2
                buf = R[uu]
                cix = CIDX[uu]

                @pl.when(u < nunits)
                def _():
                    @pl.when(u >= 2)
                    def _():
                        drain_scatter()

                    pre_ok = (u + 2 < nunits) if uu < 2 else (
                        (g < CHR // 2 - 1) & (u + 2 < nunits))

                    @pl.when(pre_ok)
                    def _():
                        issue_gather(gidx(2 * g + (uu + 2) // 2, h),
                                     R[(uu + 2) % 4])

                    wait_gather(buf)
                    scale(k, h, buf)
                    for q in range(4):
                        cix[pl.ds(q * 16, 16)] = (
                            cbig[k, pl.ds(64 * h + q * 16, 16)])
                    pltpu.async_copy(buf, acc.at[cix], ssem, add=True)

            return 0

        lax.fori_loop(0, CHR // 2, group_body, 0)
        return 0

    lax.fori_loop(0, NCHK, chunk_body, 0)
    drain_scatter()
    drain_scatter()
    plsc.subcore_barrier()

    def wb(t, _):
        sl = pl.ds(lo_own + t * ZR, ZR)

        @pl.when(c == 0)
        def _():
            pltpu.sync_copy(acc.at[sl], out0.at[sl])

        @pl.when(c == 1)
        def _():
            pltpu.sync_copy(acc.at[sl], out1.at[sl])

        return 0

    lax.fori_loop(0, ncop, wb, 0)


_spmm_call = pl.kernel(
    _spmm_body,
    out_type=(jax.ShapeDtypeStruct((N, DH), F32),
              jax.ShapeDtypeStruct((N, DH), F32)),
    mesh=_mesh(),
    scratch_types=[
        pltpu.VMEM((CHR, 128), jnp.int32),
        pltpu.VMEM((CHR, 128), jnp.int32),
        pltpu.VMEM((CHR, 128), F32),
        pltpu.VMEM((64, DH), F32),
        pltpu.VMEM((64, DH), F32),
        pltpu.VMEM((64, DH), F32),
        pltpu.VMEM((64, DH), F32),
        pltpu.VMEM((64,), jnp.int32),
        pltpu.VMEM((64,), jnp.int32),
        pltpu.VMEM((64,), jnp.int32),
        pltpu.VMEM((64,), jnp.int32),
        pltpu.VMEM_SHARED((N, DH), F32),
        pltpu.SemaphoreType.DMA,
        pltpu.SemaphoreType.DMA,
    ],
    compiler_params=_sc_params,
)


def _blk(shape, imap):
    return pl.BlockSpec(shape, imap)


_row_map = lambda i: (i, 0)
_fix_map = lambda i: (0, 0)


def _dis_body(d0_ref, d1_ref, dis_ref):
    deg = d0_ref[...][:, 0:1] + d1_ref[...][:, 0:1] + 1.0
    dis = jnp.where(deg > 0, lax.rsqrt(jnp.where(deg > 0, deg, 1.0)), 0.0)
    dis_ref[...] = jnp.broadcast_to(dis, (RB, DH))


_dis_call = pl.pallas_call(
    _dis_body,
    grid=(N // RB,),
    in_specs=[_blk((RB, 16), _row_map), _blk((RB, 16), _row_map)],
    out_specs=_blk((RB, DH), _row_map),
    out_shape=jax.ShapeDtypeStruct((N, DH), F32),
)


def _m0_body(x_ref, w_ref, dis_ref, h0_ref, h1_ref):
    dis = dis_ref[...][:, 0:1]
    h = jnp.dot(x_ref[...], w_ref[...], preferred_element_type=F32,
                precision=lax.Precision.HIGHEST) * dis
    h0_ref[...] = h[:, :DH]
    h1_ref[...] = h[:, DH:]


_m0_call = pl.pallas_call(
    _m0_body,
    grid=(N // RB,),
    in_specs=[
        _blk((RB, D), _row_map),
        _blk((D, D), _fix_map),
        _blk((RB, DH), _row_map),
    ],
    out_specs=[_blk((RB, DH), _row_map), _blk((RB, DH), _row_map)],
    out_shape=(jax.ShapeDtypeStruct((N, DH), F32),
               jax.ShapeDtypeStruct((N, DH), F32)),
)


def _asm_body(has_res, *refs):
    if has_res:
        (s0, s1, h0, h1, dis_ref, b, res, t_ref, sum_ref, sq_ref) = refs
    else:
        (s0, s1, h0, h1, dis_ref, b, t_ref, sum_ref, sq_ref) = refs
        res = None
    dis = dis_ref[...][:, 0:1]
    t = jnp.concatenate([s0[...] + h0[...], s1[...] + h1[...]], axis=1)
    t = t * dis + b[...]
    if res is not None:
        t = t + res[...]
    t_ref[...] = t

    @pl.when(pl.program_id(0) == 0)
    def _():
        sum_ref[...] = jnp.zeros_like(sum_ref)
        sq_ref[...] = jnp.zeros_like(sq_ref)

    sum_ref[...] += jnp.sum(t, axis=0, keepdims=True)
    sq_ref[...] += jnp.sum(t * t, axis=0, keepdims=True)


def _make_asm(has_res):
    in_specs = [
        _blk((RB, DH), _row_map),
        _blk((RB, DH), _row_map),
        _blk((RB, DH), _row_map),
        _blk((RB, DH), _row_map),
        _blk((RB, DH), _row_map),
        _blk((1, D), _fix_map),
    ]
    if has_res:
        in_specs.append(_blk((RB, D), _row_map))
    return pl.pallas_call(
        functools.partial(_asm_body, has_res),
        grid=(N // RB,),
        in_specs=in_specs,
        out_specs=[_blk((RB, D), _row_map), _blk((1, D), _fix_map),
                   _blk((1, D), _fix_map)],
        out_shape=(jax.ShapeDtypeStruct((N, D), F32),
                   jax.ShapeDtypeStruct((1, D), F32),
                   jax.ShapeDtypeStruct((1, D), F32)),
    )


_asm_call = _make_asm(False)
_asm_res_call = _make_asm(True)


def _bn_act(t_ref, sum_ref, sq_ref, g_ref, be_ref):
    mu = sum_ref[...] / N
    var = sq_ref[...] / N - mu * mu
    sc = lax.rsqrt(var + 1e-5) * g_ref[...]
    return jax.nn.relu((t_ref[...] - mu) * sc + be_ref[...])


def _bnmm_body(keep_act, *refs):
    if keep_act:
        (t_ref, sum_ref, sq_ref, g_ref, be_ref, w_ref, dis_ref,
         h0_ref, h1_ref, act_ref) = refs
    else:
        (t_ref, sum_ref, sq_ref, g_ref, be_ref, w_ref, dis_ref,
         h0_ref, h1_ref) = refs
        act_ref = None
    act = _bn_act(t_ref, sum_ref, sq_ref, g_ref, be_ref)
    dis = dis_ref[...][:, 0:1]
    h = jnp.dot(act, w_ref[...], preferred_element_type=F32,
                precision=lax.Precision.HIGHEST) * dis
    h0_ref[...] = h[:, :DH]
    h1_ref[...] = h[:, DH:]
    if act_ref is not None:
        act_ref[...] = act


def _make_bnmm(keep_act):
    out_specs = [_blk((RB, DH), _row_map), _blk((RB, DH), _row_map)]
    out_shape = [jax.ShapeDtypeStruct((N, DH), F32),
                 jax.ShapeDtypeStruct((N, DH), F32)]
    if keep_act:
        out_specs.append(_blk((RB, D), _row_map))
        out_shape.append(jax.ShapeDtypeStruct((N, D), F32))
    return pl.pallas_call(
        functools.partial(_bnmm_body, keep_act),
        grid=(N // RB,),
        in_specs=[
            _blk((RB, D), _row_map),
            _blk((1, D), _fix_map),
            _blk((1, D), _fix_map),
            _blk((1, D), _fix_map),
            _blk((1, D), _fix_map),
            _blk((D, D), _fix_map),
            _blk((RB, DH), _row_map),
        ],
        out_specs=out_specs,
        out_shape=tuple(out_shape),
    )


_bnmm_call = _make_bnmm(False)
_bnmm_act_call = _make_bnmm(True)


def _bnfinal_body(t_ref, sum_ref, sq_ref, g_ref, be_ref, o_ref):
    o_ref[...] = _bn_act(t_ref, sum_ref, sq_ref, g_ref, be_ref)


_bnfinal_call = pl.pallas_call(
    _bnfinal_body,
    grid=(N // RB,),
    in_specs=[
        _blk((RB, D), _row_map),
        _blk((1, D), _fix_map),
        _blk((1, D), _fix_map),
        _blk((1, D), _fix_map),
        _blk((1, D), _fix_map),
    ],
    out_specs=_blk((RB, D), _row_map),
    out_shape=jax.ShapeDtypeStruct((N, D), F32),
)


def kernel(x, edge_index, edge_attr, W0, b0, g0, be0, Wc0, bc0, gc0, bec0,
           Wc1, bc1, gc1, bec1, W1, b1, g1, be1):
    pad = ((0, ERP - ER), (0, 0))
    row2d = jnp.pad(edge_index[0].reshape(ER, 128), pad)
    col2d = jnp.pad(edge_index[1].reshape(ER, 128), pad)
    ew2d = jnp.pad(edge_attr.reshape(ER, 128), pad)
    r2 = lambda v: v.reshape(1, D)

    d0, d1 = _deg_call(col2d, ew2d)
    dis = _dis_call(d0, d1)
    h0, h1 = _m0_call(x, W0, dis)

    s0, s1 = _spmm_call(row2d, col2d, ew2d, h0, h1)
    t, sm, sq = _asm_call(s0, s1, h0, h1, dis, r2(b0))
    h0, h1, act0 = _bnmm_act_call(t, sm, sq, r2(g0), r2(be0), Wc0, dis)

    s0, s1 = _spmm_call(row2d, col2d, ew2d, h0, h1)
    t, sm, sq = _asm_res_call(s0, s1, h0, h1, dis, r2(bc0), act0)
    h0, h1 = _bnmm_call(t, sm, sq, r2(gc0), r2(bec0), Wc1, dis)

    s0, s1 = _spmm_call(row2d, col2d, ew2d, h0, h1)
    t, sm, sq = _asm_res_call(s0, s1, h0, h1, dis, r2(bc1), act0)
    h0, h1 = _bnmm_call(t, sm, sq, r2(gc1), r2(bec1), W1, dis)

    s0, s1 = _spmm_call(row2d, col2d, ew2d, h0, h1)
    t, sm, sq = _asm_call(s0, s1, h0, h1, dis, r2(b1))
    return _bnfinal_call(t, sm, sq, r2(g1), r2(be1))
